# baseline (device time: 26419 ns/iter reference)
import functools

import jax
import jax.numpy as jnp
from jax import lax
from jax.experimental import pallas as pl
from jax.experimental.pallas import tpu as pltpu

N_DEV = 16
N_TOK = 1024
N_EXP = 64
CAP = 12
EXP_PER_DEV = N_EXP // N_DEV
N_SLOT = EXP_PER_DEV * CAP
ROWS_PER_DEV = N_TOK // N_DEV


def _compute_kernel(tril, e2d, x, expert_W):
    n, d = x.shape
    h = expert_W.shape[2]

    def body(tril_ref, e_ref, x_ref, w_ref, ybuf_ref, tok_ref, keep_ref):
        my = lax.axis_index("i")
        eb = e_ref[:, :]

        iota_e = lax.broadcasted_iota(jnp.int32, (n, N_EXP), 1)
        onehot = (eb == iota_e).astype(jnp.bfloat16)

        cum = lax.dot_general(
            tril_ref[:, :], onehot,
            (((1,), (0,)), ((), ())),
            preferred_element_type=jnp.float32,
        )
        pcol = (
            jnp.sum(onehot.astype(jnp.float32) * cum, axis=1, keepdims=True)
            - 1.0
        )
        keepcol = pcol < float(CAP)
        keep_ref[:, :] = keepcol.astype(jnp.int32)

        iota_c = lax.broadcasted_iota(jnp.int32, (n, CAP), 1).astype(jnp.float32)
        peq = ((pcol == iota_c) & keepcol).astype(jnp.float32)

        iota_k = lax.broadcasted_iota(jnp.int32, (n, EXP_PER_DEV), 1)
        oh_my = (eb - my * EXP_PER_DEV == iota_k).astype(jnp.float32)

        tvals = (
            lax.broadcasted_iota(jnp.int32, (n, CAP), 0).astype(jnp.float32)
            + 1.0
        )
        myslot = lax.dot_general(
            oh_my, peq * tvals,
            (((0,), (0,)), ((), ())),
            precision=lax.Precision.HIGHEST,
        )
        tok_ref[:, :] = (jnp.rint(myslot) - 1.0).astype(jnp.int32)

        for k in range(EXP_PER_DEV):
            g_k = peq * oh_my[:, k : k + 1]
            xg_k = lax.dot_general(
                g_k, x_ref[:, :],
                (((0,), (0,)), ((), ())),
                precision=lax.Precision.HIGHEST,
            )
            ybuf_ref[k * CAP : (k + 1) * CAP, :] = jnp.dot(xg_k, w_ref[k])

    return pl.pallas_call(
        body,
        out_shape=(
            jax.ShapeDtypeStruct((N_SLOT, h), jnp.float32),
            jax.ShapeDtypeStruct((EXP_PER_DEV, CAP), jnp.int32),
            jax.ShapeDtypeStruct((n, 1), jnp.int32),
        ),
        in_specs=[
            pl.BlockSpec(memory_space=pltpu.VMEM),
            pl.BlockSpec(memory_space=pltpu.VMEM),
            pl.BlockSpec(memory_space=pltpu.VMEM),
            pl.BlockSpec(memory_space=pltpu.VMEM),
        ],
        out_specs=(
            pl.BlockSpec(memory_space=pltpu.VMEM),
            pl.BlockSpec(memory_space=pltpu.VMEM),
            pl.BlockSpec(memory_space=pltpu.VMEM),
        ),
    )(tril, e2d, x, expert_W)


def _row_scatter(ybuf, tok, keepmat, h):

    def body(ybuf_ref, tok_ref, keep_ref, out_ref, send_sem, recv_sem):
        me = lax.axis_index("i")

        out_ref[:, :] = jnp.zeros((ROWS_PER_DEV, h), jnp.float32)

        barrier_sem = pltpu.get_barrier_semaphore()
        for k in range(1, N_DEV):
            pl.semaphore_signal(
                barrier_sem, inc=1,
                device_id=((me + k) % N_DEV,),
                device_id_type=pl.DeviceIdType.MESH,
            )
        pl.semaphore_wait(barrier_sem, N_DEV - 1)

        for j in range(N_SLOT):
            t = tok_ref[j // CAP, j % CAP]

            @pl.when(t >= 0)
            def _():
                rdma = pltpu.make_async_remote_copy(
                    src_ref=ybuf_ref.at[pl.ds(j, 1)],
                    dst_ref=out_ref.at[pl.ds(lax.rem(t, ROWS_PER_DEV), 1)],
                    send_sem=send_sem,
                    recv_sem=recv_sem,
                    device_id=(lax.div(t, ROWS_PER_DEV),),
                    device_id_type=pl.DeviceIdType.MESH,
                )
                rdma.start()

        n_sent = lax.fori_loop(
            0, N_SLOT,
            lambda j, s: s + jnp.where(tok_ref[j // CAP, j % CAP] >= 0, 1, 0),
            0,
        )
        n_recv = lax.fori_loop(
            0, ROWS_PER_DEV, lambda j, s: s + keep_ref[me, j], 0
        )

        dummy = pltpu.make_async_remote_copy(
            src_ref=ybuf_ref.at[pl.ds(0, 1)],
            dst_ref=out_ref.at[pl.ds(0, 1)],
            send_sem=send_sem,
            recv_sem=recv_sem,
            device_id=(me,),
            device_id_type=pl.DeviceIdType.MESH,
        )
        lax.fori_loop(0, n_recv, lambda j, c: (dummy.wait_recv(), c)[1], 0)
        lax.fori_loop(0, n_sent, lambda j, c: (dummy.wait_send(), c)[1], 0)

        @functools.partial(
            pl.run_scoped, second_barrier=pltpu.SemaphoreType.REGULAR
        )
        def _(second_barrier):
            for k in range(1, N_DEV):
                pl.semaphore_signal(
                    second_barrier, inc=1,
                    device_id=((me + k) % N_DEV,),
                    device_id_type=pl.DeviceIdType.MESH,
                )
            pl.semaphore_wait(second_barrier, N_DEV - 1)

    return pl.pallas_call(
        body,
        out_shape=jax.ShapeDtypeStruct((ROWS_PER_DEV, h), jnp.float32),
        in_specs=[
            pl.BlockSpec(memory_space=pltpu.VMEM),
            pl.BlockSpec(memory_space=pltpu.SMEM),
            pl.BlockSpec(memory_space=pltpu.SMEM),
        ],
        out_specs=pl.BlockSpec(memory_space=pltpu.VMEM),
        scratch_shapes=[
            pltpu.SemaphoreType.DMA,
            pltpu.SemaphoreType.DMA,
        ],
        compiler_params=pltpu.CompilerParams(collective_id=0),
    )(ybuf, tok, keepmat)


def kernel(x, router_W, route_idx, expert_W):
    del router_W
    n, d = x.shape
    h = expert_W.shape[2]

    tril = jnp.tril(jnp.ones((n, n), jnp.bfloat16))
    e2d = route_idx.astype(jnp.int32)

    ybuf, tok, keep = _compute_kernel(tril, e2d, x, expert_W)
    keepmat = keep.reshape(N_DEV, ROWS_PER_DEV)
    return _row_scatter(ybuf, tok, keepmat, h)
